# baseline (speedup 1.0000x reference)
import functools

import jax
import jax.numpy as jnp
from jax import lax
from jax.experimental import pallas as pl
from jax.experimental.pallas import tpu as pltpu
from jax.experimental.pallas import tpu_sc as plsc

N_NODES = 100000
D = 128
N0 = 10000
N1 = 2000
N2 = 1024

NC = 2
NS = 16
NW = NC * NS

G_PAD = 12288
G0_PER = 256
G1_PER = 512

I2_PAD = 3072

MB = 200
NMB = N1 // MB


def _sc_gather(raw_features, src_nodes, idx_l1):
    mesh = plsc.VectorSubcoreMesh(core_axis_name="c", subcore_axis_name="s")

    @functools.partial(
        pl.kernel,
        mesh=mesh,
        out_type=jax.ShapeDtypeStruct((G_PAD, D), jnp.float32),
        scratch_types=[
            pltpu.VMEM((G1_PER,), jnp.int32),
            pltpu.VMEM((G1_PER,), jnp.int32),
            pltpu.VMEM((G1_PER, D), jnp.float32),
            pltpu.SemaphoreType.DMA,
        ],
        compiler_params=pltpu.CompilerParams(needs_layout_passes=False),
    )
    def k(raw_hbm, nodes_hbm, idx_hbm, out_hbm, idx_raw, idx_c, rows, sem):
        cid = lax.axis_index("c")
        sid = lax.axis_index("s")

        def gather_slice(base, per):
            pltpu.sync_copy(idx_hbm.at[pl.ds(base, per)],
                            idx_raw.at[pl.ds(0, per)])
            h1 = []
            for j in range(per // 16):
                v = idx_raw[pl.ds(j * 16, 16)]
                h1.append(pltpu.async_copy(nodes_hbm.at[v],
                                           idx_c.at[pl.ds(j * 16, 16)], sem))
            for h in h1:
                h.wait()
            h2 = []
            for j in range(per // 16):
                v = idx_c[pl.ds(j * 16, 16)]
                h2.append(pltpu.async_copy(raw_hbm.at[v],
                                           rows.at[pl.ds(j * 16, 16)], sem))
            for h in h2:
                h.wait()
            pltpu.sync_copy(rows.at[pl.ds(0, per)],
                            out_hbm.at[pl.ds(base, per)])

        @pl.when(cid == 0)
        def _core0():
            gather_slice(sid * G0_PER, G0_PER)

        @pl.when(cid == 1)
        def _core1():
            gather_slice(NS * G0_PER + sid * G1_PER, G1_PER)

    return k(raw_features, src_nodes, idx_l1)


def _tc_body(dif1_r, gall_r, w1_r, dif2_r, i2_r, w2_r, out_r, x1, src2):
    kk = pl.program_id(0)

    s = gall_r[pl.ds(N1, N0), :]
    agg_rows = jnp.dot(dif1_r[...], s, preferred_element_type=jnp.float32)
    dst1 = gall_r[pl.ds(kk * MB, MB), :]
    x1[pl.ds(kk * MB, MB), :] = jnp.maximum(
        jnp.dot(dst1, w1_r[pl.ds(0, D), :],
                preferred_element_type=jnp.float32)
        + jnp.dot(agg_rows, w1_r[pl.ds(D, D), :],
                  preferred_element_type=jnp.float32),
        0.0)

    @pl.when(kk == NMB - 1)
    def _final():
        x1v = x1[...]
        for b in range(5):
            idx = i2_r[pl.ds(b * 400, 400), :]
            colj = lax.broadcasted_iota(jnp.int32, (400, N1), 1)
            oh = (idx == colj).astype(jnp.float32)
            src2[pl.ds(b * 400, 400), :] = jnp.dot(
                oh, x1v, preferred_element_type=jnp.float32)
        ztop = jnp.dot(x1v, w2_r[pl.ds(0, D), :],
                       preferred_element_type=jnp.float32)
        agg2 = jnp.dot(dif2_r[...], src2[...],
                       preferred_element_type=jnp.float32)
        zbot = jnp.dot(agg2, w2_r[pl.ds(D, D), :],
                       preferred_element_type=jnp.float32)
        for b in range(4):
            idx = i2_r[pl.ds(N1 + b * 256, 256), :]
            colj = lax.broadcasted_iota(jnp.int32, (256, N1), 1)
            oh = (idx == colj).astype(jnp.float32)
            out_r[pl.ds(b * 256, 256), :] = (
                jnp.dot(oh, ztop, preferred_element_type=jnp.float32)
                + zbot[b * 256:(b + 1) * 256, :])


def _tc_main(gall, dif_mat_l1, w1, dif_mat_l2, i2, w2, interpret=False):
    return pl.pallas_call(
        _tc_body,
        grid=(NMB,),
        in_specs=[
            pl.BlockSpec((MB, N0), lambda k: (k, 0)),
            pl.BlockSpec((G_PAD, D), lambda k: (0, 0)),
            pl.BlockSpec((2 * D, D), lambda k: (0, 0)),
            pl.BlockSpec((N2, N1), lambda k: (0, 0)),
            pl.BlockSpec((I2_PAD, 1), lambda k: (0, 0)),
            pl.BlockSpec((2 * D, D), lambda k: (0, 0)),
        ],
        out_specs=pl.BlockSpec((N2, D), lambda k: (0, 0)),
        out_shape=jax.ShapeDtypeStruct((N2, D), jnp.float32),
        scratch_shapes=[
            pltpu.VMEM((N1, D), jnp.float32),
            pltpu.VMEM((N1, D), jnp.float32),
        ],
        compiler_params=pltpu.CompilerParams(
            dimension_semantics=("arbitrary",)),
        interpret=interpret,
    )(dif_mat_l1, gall, w1, dif_mat_l2, i2, w2)


def kernel(raw_features, src_nodes, dstsrc2src_l1, dstsrc2dst_l1, dif_mat_l1,
           dstsrc2src_l2, dstsrc2dst_l2, dif_mat_l2, w1, w2):
    idx_l1 = jnp.pad(
        jnp.concatenate([dstsrc2dst_l1.astype(jnp.int32),
                         dstsrc2src_l1.astype(jnp.int32)]),
        (0, G_PAD - N1 - N0))
    gall = _sc_gather(raw_features, src_nodes.astype(jnp.int32), idx_l1)
    i2 = jnp.pad(
        jnp.concatenate([dstsrc2src_l2.astype(jnp.int32),
                         dstsrc2dst_l2.astype(jnp.int32)]),
        (0, I2_PAD - N1 - N2)).reshape(I2_PAD, 1)
    return _tc_main(gall, dif_mat_l1, w1, dif_mat_l2, i2, w2)


# speedup vs baseline: 1.0204x; 1.0204x over previous
import functools

import jax
import jax.numpy as jnp
from jax import lax
from jax.experimental import pallas as pl
from jax.experimental.pallas import tpu as pltpu
from jax.experimental.pallas import tpu_sc as plsc

N_NODES = 100000
D = 128
N0 = 10000
N1 = 2000
N2 = 1024

NC = 2
NS = 16
NW = NC * NS

G_PAD = 12288
G_PER = G_PAD // NW
NV = G_PER // 16

I2_PAD = 3072

MB = 200
NMB = N1 // MB


def _sc_gather(raw_features, src_nodes, idx_l1):
    mesh = plsc.VectorSubcoreMesh(core_axis_name="c", subcore_axis_name="s")

    @functools.partial(
        pl.kernel,
        mesh=mesh,
        out_type=jax.ShapeDtypeStruct((G_PAD, D), jnp.float32),
        scratch_types=[
            pltpu.VMEM((G_PER,), jnp.int32),
            pltpu.VMEM((G_PER,), jnp.int32),
            pltpu.VMEM((G_PER, D), jnp.float32),
            pltpu.SemaphoreType.DMA,
        ],
        compiler_params=pltpu.CompilerParams(needs_layout_passes=False),
    )
    def k(raw_hbm, nodes_hbm, idx_hbm, out_hbm, idx_raw, idx_c, rows, sem):
        wid = lax.axis_index("c") * NS + lax.axis_index("s")
        pltpu.sync_copy(idx_hbm.at[pl.ds(wid * G_PER, G_PER)], idx_raw)
        h1 = []
        for j in range(NV):
            v = idx_raw[pl.ds(j * 16, 16)]
            h1.append(pltpu.async_copy(nodes_hbm.at[v],
                                       idx_c.at[pl.ds(j * 16, 16)], sem))
        for h in h1:
            h.wait()
        h2 = []
        for j in range(NV):
            v = idx_c[pl.ds(j * 16, 16)]
            h2.append(pltpu.async_copy(raw_hbm.at[v],
                                       rows.at[pl.ds(j * 16, 16)], sem))
        for h in h2:
            h.wait()
        pltpu.sync_copy(rows, out_hbm.at[pl.ds(wid * G_PER, G_PER)])

    return k(raw_features, src_nodes, idx_l1)


def _tc_body(dif1_r, gall_r, w1_r, dif2_r, i2_r, w2_r, out_r, x1, src2):
    kk = pl.program_id(0)

    s = gall_r[pl.ds(N1, N0), :]
    agg_rows = jnp.dot(dif1_r[...], s, preferred_element_type=jnp.float32)
    dst1 = gall_r[pl.ds(kk * MB, MB), :]
    x1[pl.ds(kk * MB, MB), :] = jnp.maximum(
        jnp.dot(dst1, w1_r[pl.ds(0, D), :],
                preferred_element_type=jnp.float32)
        + jnp.dot(agg_rows, w1_r[pl.ds(D, D), :],
                  preferred_element_type=jnp.float32),
        0.0)

    @pl.when(kk == NMB - 1)
    def _final():
        x1v = x1[...]
        for b in range(5):
            idx = i2_r[pl.ds(b * 400, 400), :]
            colj = lax.broadcasted_iota(jnp.int32, (400, N1), 1)
            oh = (idx == colj).astype(jnp.float32)
            src2[pl.ds(b * 400, 400), :] = jnp.dot(
                oh, x1v, preferred_element_type=jnp.float32)
        ztop = jnp.dot(x1v, w2_r[pl.ds(0, D), :],
                       preferred_element_type=jnp.float32)
        agg2 = jnp.dot(dif2_r[...], src2[...],
                       preferred_element_type=jnp.float32)
        zbot = jnp.dot(agg2, w2_r[pl.ds(D, D), :],
                       preferred_element_type=jnp.float32)
        for b in range(4):
            idx = i2_r[pl.ds(N1 + b * 256, 256), :]
            colj = lax.broadcasted_iota(jnp.int32, (256, N1), 1)
            oh = (idx == colj).astype(jnp.float32)
            out_r[pl.ds(b * 256, 256), :] = (
                jnp.dot(oh, ztop, preferred_element_type=jnp.float32)
                + zbot[b * 256:(b + 1) * 256, :])


def _tc_main(gall, dif_mat_l1, w1, dif_mat_l2, i2, w2, interpret=False):
    return pl.pallas_call(
        _tc_body,
        grid=(NMB,),
        in_specs=[
            pl.BlockSpec((MB, N0), lambda k: (k, 0)),
            pl.BlockSpec((G_PAD, D), lambda k: (0, 0)),
            pl.BlockSpec((2 * D, D), lambda k: (0, 0)),
            pl.BlockSpec((N2, N1), lambda k: (0, 0)),
            pl.BlockSpec((I2_PAD, 1), lambda k: (0, 0)),
            pl.BlockSpec((2 * D, D), lambda k: (0, 0)),
        ],
        out_specs=pl.BlockSpec((N2, D), lambda k: (0, 0)),
        out_shape=jax.ShapeDtypeStruct((N2, D), jnp.float32),
        scratch_shapes=[
            pltpu.VMEM((N1, D), jnp.float32),
            pltpu.VMEM((N1, D), jnp.float32),
        ],
        compiler_params=pltpu.CompilerParams(
            dimension_semantics=("arbitrary",)),
        interpret=interpret,
    )(dif_mat_l1, gall, w1, dif_mat_l2, i2, w2)


def kernel(raw_features, src_nodes, dstsrc2src_l1, dstsrc2dst_l1, dif_mat_l1,
           dstsrc2src_l2, dstsrc2dst_l2, dif_mat_l2, w1, w2):
    idx_l1 = jnp.pad(
        jnp.concatenate([dstsrc2dst_l1.astype(jnp.int32),
                         dstsrc2src_l1.astype(jnp.int32)]),
        (0, G_PAD - N1 - N0))
    gall = _sc_gather(raw_features, src_nodes.astype(jnp.int32), idx_l1)
    i2 = jnp.pad(
        jnp.concatenate([dstsrc2src_l2.astype(jnp.int32),
                         dstsrc2dst_l2.astype(jnp.int32)]),
        (0, I2_PAD - N1 - N2)).reshape(I2_PAD, 1)
    return _tc_main(gall, dif_mat_l1, w1, dif_mat_l2, i2, w2)
